# baseline (speedup 1.0000x reference)
import jax
import jax.numpy as jnp
from jax.experimental import pallas as pl
from jax.experimental.pallas import tpu as pltpu

_N = 10000
_DF = 128
_DH = 64
_DE = 16
_B1 = 400
_B3 = 400


def _p1(feats_ref, w1_ref, w2_ref, adj_ref, y_ref, x1_ref):
    @pl.when(pl.program_id(0) == 0)
    def _():
        x1_ref[...] = jnp.dot(feats_ref[...], w1_ref[...],
                              preferred_element_type=jnp.float32)
    h = jnp.dot(adj_ref[...], x1_ref[...], preferred_element_type=jnp.float32)
    h = jnp.maximum(h, 0.0)
    y_ref[...] = jnp.dot(h, w2_ref[...], preferred_element_type=jnp.float32)


def _p2(adj_ref, y_ref, z_ref):
    z = jnp.dot(adj_ref[...], y_ref[...], preferred_element_type=jnp.float32)
    z_ref[...] = jnp.maximum(z, 0.0)


def _p3(zi_ref, zall_ref, out_ref):
    out_ref[...] = jax.lax.dot_general(
        zi_ref[...], zall_ref[...],
        (((1,), (1,)), ((), ())),
        preferred_element_type=jnp.float32)


def kernel(feats, adj, W1, W2):
    nb1 = _N // _B1
    y = pl.pallas_call(
        _p1,
        grid=(nb1,),
        in_specs=[
            pl.BlockSpec((_N, _DF), lambda i: (0, 0)),
            pl.BlockSpec((_DF, _DH), lambda i: (0, 0)),
            pl.BlockSpec((_DH, _DE), lambda i: (0, 0)),
            pl.BlockSpec((_B1, _N), lambda i: (i, 0)),
        ],
        out_specs=pl.BlockSpec((_B1, _DE), lambda i: (i, 0)),
        out_shape=jax.ShapeDtypeStruct((_N, _DE), jnp.float32),
        scratch_shapes=[pltpu.VMEM((_N, _DH), jnp.float32)],
        compiler_params=pltpu.CompilerParams(
            dimension_semantics=("arbitrary",)),
    )(feats, W1, W2, adj)

    z = pl.pallas_call(
        _p2,
        grid=(nb1,),
        in_specs=[
            pl.BlockSpec((_B1, _N), lambda i: (i, 0)),
            pl.BlockSpec((_N, _DE), lambda i: (0, 0)),
        ],
        out_specs=pl.BlockSpec((_B1, _DE), lambda i: (i, 0)),
        out_shape=jax.ShapeDtypeStruct((_N, _DE), jnp.float32),
        compiler_params=pltpu.CompilerParams(
            dimension_semantics=("arbitrary",)),
    )(adj, y)

    nb3 = _N // _B3
    out = pl.pallas_call(
        _p3,
        grid=(nb3,),
        in_specs=[
            pl.BlockSpec((_B3, _DE), lambda i: (i, 0)),
            pl.BlockSpec((_N, _DE), lambda i: (0, 0)),
        ],
        out_specs=pl.BlockSpec((_B3, _N), lambda i: (i, 0)),
        out_shape=jax.ShapeDtypeStruct((_N, _N), jnp.float32),
        compiler_params=pltpu.CompilerParams(
            dimension_semantics=("arbitrary",)),
    )(z, z)
    return out
